# baseline (speedup 1.0000x reference)
import functools

import jax
import jax.numpy as jnp
from jax import lax
from jax.experimental import pallas as pl
from jax.experimental.pallas import tpu as pltpu
from jax.experimental.pallas import tpu_sc as plsc

N = 10000
E = 320000
D = 128
NC = 2
NS = 16
NW = NC * NS
EPT = E // NW
CHUNK = 128
NFULL = EPT // CHUNK
REM = EPT - NFULL * CHUNK
RPT = 624
TAIL = N - NS * RPT

NB = 3
DB = 6
UNROLL = 13


def _sc_hop(h, src, dst, zeros):
    mesh = plsc.VectorSubcoreMesh(core_axis_name="c", subcore_axis_name="s",
                                  num_cores=NC, num_subcores=NS)

    @functools.partial(
        pl.kernel,
        mesh=mesh,
        out_type=jax.ShapeDtypeStruct((NC, N, D), jnp.float32),
        scratch_types=(
            [pltpu.VMEM((CHUNK, D), jnp.float32) for _ in range(NB)]
            + [pltpu.VMEM((CHUNK,), jnp.int32) for _ in range(DB)]
            + [pltpu.VMEM((CHUNK,), jnp.int32) for _ in range(DB)]
            + [pltpu.VMEM((REM,), jnp.int32)]
            + [pltpu.VMEM((REM,), jnp.int32)]
            + [pltpu.VMEM_SHARED((N, D), jnp.float32)]
            + [pltpu.SemaphoreType.DMA for _ in range(2 * DB + 2 * NB + 1)]
        ),
    )
    def hop(h_hbm, src_hbm, dst_hbm, z_hbm, out_hbm, *refs):
        rbs = refs[0:NB]
        sbs = refs[NB:NB + DB]
        dbs = refs[NB + DB:NB + 2 * DB]
        src_r = refs[NB + 2 * DB]
        dst_r = refs[NB + 2 * DB + 1]
        agg = refs[NB + 2 * DB + 2]
        sems = refs[NB + 2 * DB + 3:]
        isems = sems[0:DB]
        jsems = sems[DB:2 * DB]
        gsems = sems[2 * DB:2 * DB + NB]
        ssems = sems[2 * DB + NB:2 * DB + 2 * NB]
        rsem = sems[2 * DB + 2 * NB]

        c = lax.axis_index("c")
        s = lax.axis_index("s")
        wid = s * NC + c

        row0 = s * RPT
        pltpu.sync_copy(z_hbm.at[pl.ds(row0, RPT)], agg.at[pl.ds(row0, RPT)])

        @pl.when(s == NS - 1)
        def _zero_tail():
            pltpu.sync_copy(z_hbm.at[pl.ds(NS * RPT, TAIL)],
                            agg.at[pl.ds(NS * RPT, TAIL)])
        plsc.subcore_barrier()

        def idx_issue(j, k):
            base = wid * EPT + j * CHUNK
            a = pltpu.async_copy(src_hbm.at[pl.ds(base, CHUNK)],
                                 sbs[k % DB], isems[k % DB])
            b = pltpu.async_copy(dst_hbm.at[pl.ds(base, CHUNK)],
                                 dbs[k % DB], jsems[k % DB])
            return (a, b)

        def gather(k, p):
            return pltpu.async_copy(h_hbm.at[sbs[k % DB]], rbs[p], gsems[p])

        def scat(k, p):
            return pltpu.async_copy(rbs[p], agg.at[dbs[k % DB]], ssems[p],
                                    add=True)

        def emit(j0, n):
            ii = [None] * n
            gd = [None] * n
            sd = [None] * n
            for k in range(min(n, DB)):
                ii[k] = idx_issue(j0 + k, k)
            for k in range(n):
                if k >= NB:
                    sd[k - NB].wait()
                    nk = k - NB + DB
                    if nk < n:
                        ii[nk] = idx_issue(j0 + nk, nk)
                ii[k][0].wait()
                gd[k] = gather(k, k % NB)
                if k >= 1:
                    gd[k - 1].wait()
                    ii[k - 1][1].wait()
                    sd[k - 1] = scat(k - 1, (k - 1) % NB)
            gd[n - 1].wait()
            ii[n - 1][1].wait()
            sd[n - 1] = scat(n - 1, (n - 1) % NB)
            for k in range(max(0, n - NB), n):
                sd[k].wait()

        def body(i, carry):
            emit(UNROLL * i, UNROLL)
            return carry
        lax.fori_loop(0, NFULL // UNROLL, body, 0)

        rbase = wid * EPT + NFULL * CHUNK
        pltpu.sync_copy(src_hbm.at[pl.ds(rbase, REM)], src_r)
        ir = pltpu.async_copy(dst_hbm.at[pl.ds(rbase, REM)], dst_r, rsem)
        gr = pltpu.async_copy(h_hbm.at[src_r], rbs[0].at[pl.ds(0, REM)], rsem)
        ir.wait()
        gr.wait()
        pltpu.sync_copy(rbs[0].at[pl.ds(0, REM)], agg.at[dst_r], add=True)

        plsc.subcore_barrier()
        pltpu.sync_copy(agg.at[pl.ds(row0, RPT)],
                        out_hbm.at[c, pl.ds(row0, RPT)])

        @pl.when(s == NS - 1)
        def _copy_tail():
            pltpu.sync_copy(agg.at[pl.ds(NS * RPT, TAIL)],
                            out_hbm.at[c, pl.ds(NS * RPT, TAIL)])

    return hop(h, src, dst, zeros)


_DOT = dict(preferred_element_type=jnp.float32,
            precision=lax.Precision.HIGHEST)
_BR = 1000


def _tc_mid(p, h, wr, wt):
    def body(p_ref, h_ref, wr_ref, wt_ref, o_ref):
        agg = p_ref[0] + p_ref[1]
        y = lax.dot_general(agg, wr_ref[...], (((1,), (1,)), ((), ())), **_DOT)
        y = y + lax.dot_general(h_ref[...], wt_ref[...],
                                (((1,), (1,)), ((), ())), **_DOT)
        o_ref[...] = jnp.maximum(y, 0.0)

    return pl.pallas_call(
        body,
        grid=(N // _BR,),
        in_specs=[
            pl.BlockSpec((NC, _BR, D), lambda i: (0, i, 0)),
            pl.BlockSpec((_BR, D), lambda i: (i, 0)),
            pl.BlockSpec((D, D), lambda i: (0, 0)),
            pl.BlockSpec((D, D), lambda i: (0, 0)),
        ],
        out_specs=pl.BlockSpec((_BR, D), lambda i: (i, 0)),
        out_shape=jax.ShapeDtypeStruct((N, D), jnp.float32),
    )(p, h, wr, wt)


def _tc_final(p, h, wr_s, wt_s):
    ksel = wr_s.shape[0]

    def body(p_ref, h_ref, wr_ref, wt_ref, o_ref):
        agg = p_ref[0] + p_ref[1]
        y = lax.dot_general(agg, wr_ref[...], (((1,), (1,)), ((), ())), **_DOT)
        y = y + lax.dot_general(h_ref[...], wt_ref[...],
                                (((1,), (1,)), ((), ())), **_DOT)
        o_ref[...] = jnp.sum(jnp.maximum(y, 0.0), axis=1, keepdims=True)

    return pl.pallas_call(
        body,
        grid=(N // _BR,),
        in_specs=[
            pl.BlockSpec((NC, _BR, D), lambda i: (0, i, 0)),
            pl.BlockSpec((_BR, D), lambda i: (i, 0)),
            pl.BlockSpec((ksel, D), lambda i: (0, 0)),
            pl.BlockSpec((ksel, D), lambda i: (0, 0)),
        ],
        out_specs=pl.BlockSpec((_BR, 1), lambda i: (i, 0)),
        out_shape=jax.ShapeDtypeStruct((N, 1), jnp.float32),
    )(p, h, wr_s, wt_s)


def kernel(x, edge_index, batch, W_rel, W_root):
    del batch
    src = edge_index[0]
    dst = edge_index[1]
    step = 4
    wr_s = W_rel[step - 1::step]
    wt_s = W_root[step - 1::step]

    zeros = jnp.zeros((N, D), jnp.float32)
    p1 = _sc_hop(x, src, dst, zeros)
    h1 = _tc_mid(p1, x, W_rel, W_root)
    p2 = _sc_hop(h1, src, dst, zeros)
    out = _tc_final(p2, h1, wr_s, wt_s)
    return out[:, 0]


# speedup vs baseline: 1.0216x; 1.0216x over previous
import functools

import jax
import jax.numpy as jnp
from jax import lax
from jax.experimental import pallas as pl
from jax.experimental.pallas import tpu as pltpu
from jax.experimental.pallas import tpu_sc as plsc

N = 10000
E = 320000
D = 128
NC = 2
NS = 16
NW = NC * NS
EPT = E // NW
CHUNK = 104
NFULL = EPT // CHUNK
REM = EPT - NFULL * CHUNK
RPT = 624
TAIL = N - NS * RPT

NB = 3
DB = 6
UNROLL = 12


def _sc_hop(h, src, dst, zeros):
    mesh = plsc.VectorSubcoreMesh(core_axis_name="c", subcore_axis_name="s",
                                  num_cores=NC, num_subcores=NS)

    @functools.partial(
        pl.kernel,
        mesh=mesh,
        out_type=jax.ShapeDtypeStruct((NC, N, D), jnp.float32),
        scratch_types=(
            [pltpu.VMEM((EPT,), jnp.int32)]
            + [pltpu.VMEM((CHUNK, D), jnp.float32) for _ in range(NB)]
            + [pltpu.VMEM((CHUNK,), jnp.int32) for _ in range(DB)]
            + [pltpu.VMEM((REM,), jnp.int32)]
            + [pltpu.VMEM_SHARED((N, D), jnp.float32)]
            + [pltpu.SemaphoreType.DMA for _ in range(1 + DB + 2 * NB + 1)]
        ),
    )
    def hop(h_hbm, src_hbm, dst_hbm, z_hbm, out_hbm, *refs):
        srcall = refs[0]
        rbs = refs[1:1 + NB]
        dbs = refs[1 + NB:1 + NB + DB]
        dst_r = refs[1 + NB + DB]
        agg = refs[1 + NB + DB + 1]
        sems = refs[1 + NB + DB + 2:]
        lsem = sems[0]
        isems = sems[1:1 + DB]
        gsems = sems[1 + DB:1 + DB + NB]
        ssems = sems[1 + DB + NB:1 + DB + 2 * NB]
        rsem = sems[1 + DB + 2 * NB]

        c = lax.axis_index("c")
        s = lax.axis_index("s")
        wid = s * NC + c

        lsrc = pltpu.async_copy(src_hbm.at[pl.ds(wid * EPT, EPT)], srcall,
                                lsem)
        row0 = s * RPT
        pltpu.sync_copy(z_hbm.at[pl.ds(row0, RPT)], agg.at[pl.ds(row0, RPT)])

        @pl.when(s == NS - 1)
        def _zero_tail():
            pltpu.sync_copy(z_hbm.at[pl.ds(NS * RPT, TAIL)],
                            agg.at[pl.ds(NS * RPT, TAIL)])
        lsrc.wait()
        plsc.subcore_barrier()

        def idx_issue(j, k):
            base = wid * EPT + j * CHUNK
            return pltpu.async_copy(dst_hbm.at[pl.ds(base, CHUNK)],
                                    dbs[k % DB], isems[k % DB])

        def gather(j, p):
            return pltpu.async_copy(
                h_hbm.at[srcall.at[pl.ds(j * CHUNK, CHUNK)]], rbs[p], gsems[p])

        def scat(k, p):
            return pltpu.async_copy(rbs[p], agg.at[dbs[k % DB]], ssems[p],
                                    add=True)

        def emit(j0, n):
            ii = [None] * n
            gd = [None] * n
            sd = [None] * n
            for k in range(min(n, DB)):
                ii[k] = idx_issue(j0 + k, k)
            for k in range(n):
                if k >= NB:
                    sd[k - NB].wait()
                    nk = k - NB + DB
                    if nk < n:
                        ii[nk] = idx_issue(j0 + nk, nk)
                gd[k] = gather(j0 + k, k % NB)
                if k >= 1:
                    gd[k - 1].wait()
                    ii[k - 1].wait()
                    sd[k - 1] = scat(k - 1, (k - 1) % NB)
            gd[n - 1].wait()
            ii[n - 1].wait()
            sd[n - 1] = scat(n - 1, (n - 1) % NB)
            for k in range(max(0, n - NB), n):
                sd[k].wait()

        def body(i, carry):
            emit(UNROLL * i, UNROLL)
            return carry
        lax.fori_loop(0, NFULL // UNROLL, body, 0)

        rbase = NFULL * CHUNK
        ir = pltpu.async_copy(dst_hbm.at[pl.ds(wid * EPT + rbase, REM)],
                              dst_r, rsem)
        gr = pltpu.async_copy(
            h_hbm.at[srcall.at[pl.ds(rbase, REM)]], rbs[0].at[pl.ds(0, REM)],
            rsem)
        ir.wait()
        gr.wait()
        pltpu.sync_copy(rbs[0].at[pl.ds(0, REM)], agg.at[dst_r], add=True)

        plsc.subcore_barrier()
        pltpu.sync_copy(agg.at[pl.ds(row0, RPT)],
                        out_hbm.at[c, pl.ds(row0, RPT)])

        @pl.when(s == NS - 1)
        def _copy_tail():
            pltpu.sync_copy(agg.at[pl.ds(NS * RPT, TAIL)],
                            out_hbm.at[c, pl.ds(NS * RPT, TAIL)])

    return hop(h, src, dst, zeros)


_DOT = dict(preferred_element_type=jnp.float32,
            precision=lax.Precision.HIGHEST)
_BR = 1000


def _tc_mid(p, h, wr, wt):
    def body(p_ref, h_ref, wr_ref, wt_ref, o_ref):
        agg = p_ref[0] + p_ref[1]
        y = lax.dot_general(agg, wr_ref[...], (((1,), (1,)), ((), ())), **_DOT)
        y = y + lax.dot_general(h_ref[...], wt_ref[...],
                                (((1,), (1,)), ((), ())), **_DOT)
        o_ref[...] = jnp.maximum(y, 0.0)

    return pl.pallas_call(
        body,
        grid=(N // _BR,),
        in_specs=[
            pl.BlockSpec((NC, _BR, D), lambda i: (0, i, 0)),
            pl.BlockSpec((_BR, D), lambda i: (i, 0)),
            pl.BlockSpec((D, D), lambda i: (0, 0)),
            pl.BlockSpec((D, D), lambda i: (0, 0)),
        ],
        out_specs=pl.BlockSpec((_BR, D), lambda i: (i, 0)),
        out_shape=jax.ShapeDtypeStruct((N, D), jnp.float32),
    )(p, h, wr, wt)


def _tc_final(p, h, wr_s, wt_s):
    ksel = wr_s.shape[0]

    def body(p_ref, h_ref, wr_ref, wt_ref, o_ref):
        agg = p_ref[0] + p_ref[1]
        y = lax.dot_general(agg, wr_ref[...], (((1,), (1,)), ((), ())), **_DOT)
        y = y + lax.dot_general(h_ref[...], wt_ref[...],
                                (((1,), (1,)), ((), ())), **_DOT)
        o_ref[...] = jnp.sum(jnp.maximum(y, 0.0), axis=1, keepdims=True)

    return pl.pallas_call(
        body,
        grid=(N // _BR,),
        in_specs=[
            pl.BlockSpec((NC, _BR, D), lambda i: (0, i, 0)),
            pl.BlockSpec((_BR, D), lambda i: (i, 0)),
            pl.BlockSpec((ksel, D), lambda i: (0, 0)),
            pl.BlockSpec((ksel, D), lambda i: (0, 0)),
        ],
        out_specs=pl.BlockSpec((_BR, 1), lambda i: (i, 0)),
        out_shape=jax.ShapeDtypeStruct((N, 1), jnp.float32),
    )(p, h, wr_s, wt_s)


def kernel(x, edge_index, batch, W_rel, W_root):
    del batch
    src = edge_index[0]
    dst = edge_index[1]
    step = 4
    wr_s = W_rel[step - 1::step]
    wt_s = W_root[step - 1::step]

    zeros = jnp.zeros((N, D), jnp.float32)
    p1 = _sc_hop(x, src, dst, zeros)
    h1 = _tc_mid(p1, x, W_rel, W_root)
    p2 = _sc_hop(h1, src, dst, zeros)
    out = _tc_final(p2, h1, wr_s, wt_s)
    return out[:, 0]
